# Initial kernel scaffold; baseline (speedup 1.0000x reference)
import jax
import jax.numpy as jnp
from jax.experimental import pallas as pl


def kernel(ctc_prob, g, c):
    raise NotImplementedError("write your pallas kernel here")


# speedup vs baseline: 1.3012x; 1.3012x over previous
import functools

import jax
import jax.numpy as jnp
from jax import lax
from jax.experimental import pallas as pl
from jax.experimental.pallas import tpu as pltpu
from jax.experimental.pallas import tpu_sc as plsc

NEG_INF = -1e10
EOS_ID = 1
T = 1024
V = 8000
N = 256
GLEN = 9
NC = 2
NS = 16
NW = NC * NS
T_PER = T // NW
ROWS_PER_W = T_PER * N // 128


def _lse_body(x_ref, lse_ref, blank_ref):
    x = x_ref[...]
    m = jnp.max(x, axis=1, keepdims=True)
    s = jnp.sum(jnp.exp(x - m), axis=1, keepdims=True)
    lse_ref[...] = m + jnp.log(s)
    blank_ref[...] = x[:, V - 1:V]


def _combine_body(g_ref, lse_ref, blank_ref, c_ref, o_ref):
    lse = lse_ref[...]
    blank_lp = blank_ref[...] - lse
    bl_b = jnp.broadcast_to(blank_lp, (T, N))
    row = lax.broadcasted_iota(jnp.int32, (T, T), 0)
    col = lax.broadcasted_iota(jnp.int32, (T, T), 1)
    tri = jnp.where(col < row, 1.0, 0.0).astype(jnp.float32)
    gb_prev = jnp.dot(tri, bl_b, preferred_element_type=jnp.float32)
    la = jnp.maximum(gb_prev, NEG_INF) + jnp.log(
        1.0 + jnp.exp(-jnp.abs(gb_prev - NEG_INF)))
    x = la - jnp.broadcast_to(lse, (T, N)) + g_ref[...]
    trow = lax.broadcasted_iota(jnp.int32, (T, N), 0)
    x = jnp.where(trow >= GLEN, x, NEG_INF)
    m = jnp.max(x, axis=0, keepdims=True)
    s = jnp.sum(jnp.exp(x - m), axis=0, keepdims=True)
    score = m + jnp.log(s)
    gb_full = jnp.sum(blank_lp)
    eos_val = jnp.maximum(gb_full, NEG_INF) + jnp.log(
        1.0 + jnp.exp(-jnp.abs(gb_full - NEG_INF)))
    o_ref[...] = jnp.where(c_ref[...] == EOS_ID, eos_val, score)


def _sc_gather_body(flat_hbm, c_hbm, out_hbm, c_v, idx_v, vals_v, sem):
    wid = lax.axis_index("s") * NC + lax.axis_index("c")
    t0 = wid * T_PER
    pltpu.sync_copy(c_hbm, c_v)

    def build(tl, carry):
        base = (t0 + tl) * V
        for half in range(2):
            r = tl * 2 + half
            for sub in range(8):
                cc = c_v[pl.ds(half * 128 + sub * 16, 16)]
                idx_v[r, pl.ds(sub * 16, 16)] = cc + base
        return carry

    lax.fori_loop(0, T_PER, build, 0)

    def fire(gidx, carry):
        cps = [
            pltpu.async_copy(
                flat_hbm.at[idx_v.at[gidx * 16 + i]],
                vals_v.at[gidx * 16 + i], sem)
            for i in range(16)
        ]
        for cp in cps:
            cp.wait()
        return carry

    lax.fori_loop(0, ROWS_PER_W // 16, fire, 0)

    pltpu.sync_copy(vals_v, out_hbm.at[pl.ds(wid * ROWS_PER_W, ROWS_PER_W)])


@functools.lru_cache(maxsize=1)
def _sc_gather():
    return pl.kernel(
        _sc_gather_body,
        mesh=plsc.VectorSubcoreMesh(core_axis_name="c", subcore_axis_name="s"),
        out_type=jax.ShapeDtypeStruct((NW * ROWS_PER_W, 128), jnp.float32),
        scratch_types=[
            pltpu.VMEM((N,), jnp.int32),
            pltpu.VMEM((ROWS_PER_W, 128), jnp.int32),
            pltpu.VMEM((ROWS_PER_W, 128), jnp.float32),
            pltpu.SemaphoreType.DMA,
        ],
    )


def kernel(ctc_prob, g, c):
    del g
    flat = jnp.reshape(ctc_prob, (T * V,))
    gathered = _sc_gather()(flat, c)
    lse, blank = pl.pallas_call(
        _lse_body,
        grid=(T // 128,),
        in_specs=[pl.BlockSpec((128, V), lambda i: (i, 0))],
        out_specs=[pl.BlockSpec((128, 1), lambda i: (i, 0)),
                   pl.BlockSpec((128, 1), lambda i: (i, 0))],
        out_shape=[jax.ShapeDtypeStruct((T, 1), jnp.float32),
                   jax.ShapeDtypeStruct((T, 1), jnp.float32)],
    )(ctc_prob)
    out = pl.pallas_call(
        _combine_body,
        out_shape=jax.ShapeDtypeStruct((1, N), jnp.float32),
    )(gathered.reshape(T, N), lse, blank, c.reshape(1, N))
    return out.reshape(N // 16, 16)
